# Initial kernel scaffold; baseline (speedup 1.0000x reference)
import functools

import jax
import jax.numpy as jnp
from jax import lax
from jax.experimental import pallas as pl
from jax.experimental.pallas import tpu as pltpu
from jax.experimental.pallas import tpu_sc as plsc

N = 10000
E = 320000
D = 128
C = 40
G = 64

NP = 10240
EP = 327680
NW = 32
NCHUNK = 80
ROWS_PER_TILE = NP // 16

_HIGHEST = jax.lax.Precision.HIGHEST


@functools.lru_cache(maxsize=None)
def _sc_mesh():
    return plsc.VectorSubcoreMesh(core_axis_name="c", subcore_axis_name="s")


def _deg_body(dst_hbm, out_hbm, idx_d, ones_v, zeros_v, acc):
    c = lax.axis_index("c")
    s = lax.axis_index("s")
    wid = s * 2 + c
    pltpu.sync_copy(dst_hbm.at[wid], idx_d)
    for k in range(8):
        ones_v[pl.ds(k * 16, 16)] = jnp.ones((16,), jnp.float32)

    def _zfill(t, carry):
        zeros_v[pl.ds(t * 16, 16)] = jnp.zeros((16,), jnp.float32)
        return carry

    lax.fori_loop(0, ROWS_PER_TILE // 16, _zfill, 0)
    pltpu.sync_copy(zeros_v, acc.at[pl.ds(s * ROWS_PER_TILE, ROWS_PER_TILE)])
    plsc.subcore_barrier()

    def _scat(j, carry):
        pltpu.sync_copy(ones_v, acc.at[idx_d.at[j]], add=True)
        return carry

    lax.fori_loop(0, NCHUNK, _scat, 0)
    plsc.subcore_barrier()
    pltpu.sync_copy(acc.at[pl.ds(s * ROWS_PER_TILE, ROWS_PER_TILE)],
                    out_hbm.at[c, pl.ds(s * ROWS_PER_TILE, ROWS_PER_TILE)])


@functools.lru_cache(maxsize=None)
def _deg_kernel():
    return pl.kernel(
        _deg_body,
        out_type=jax.ShapeDtypeStruct((2, NP), jnp.float32),
        mesh=_sc_mesh(),
        scratch_types=[
            pltpu.VMEM((NCHUNK, 128), jnp.int32),
            pltpu.VMEM((128,), jnp.float32),
            pltpu.VMEM((ROWS_PER_TILE,), jnp.float32),
            pltpu.VMEM_SHARED((NP,), jnp.float32),
        ],
    )


def _scatter_body(p_hbm, src_hbm, dst_hbm, out_hbm,
                  idx_s, idx_d, rows, acc, sem0, sem1):
    c = lax.axis_index("c")
    s = lax.axis_index("s")
    wid = s * 2 + c
    pltpu.sync_copy(src_hbm.at[wid], idx_s)
    pltpu.sync_copy(dst_hbm.at[wid], idx_d)

    def _zrow(r, carry):
        for k in range(8):
            rows[0, r, pl.ds(k * 16, 16)] = jnp.zeros((16,), jnp.float32)
        return carry

    lax.fori_loop(0, 128, _zrow, 0)

    def _zcopy(k, carry):
        pltpu.sync_copy(rows.at[0],
                        acc.at[pl.ds((s * 5 + k) * 128, 128)])
        return carry

    lax.fori_loop(0, 5, _zcopy, 0)
    plsc.subcore_barrier()

    pltpu.async_copy(p_hbm.at[idx_s.at[0]], rows.at[0], sem0)

    def _main(t, carry):
        j = t * 2
        pltpu.async_copy(p_hbm.at[idx_s.at[j + 1]], rows.at[1], sem1)
        pltpu.make_async_copy(p_hbm.at[idx_s.at[j]], rows.at[0], sem0).wait()
        pltpu.sync_copy(rows.at[0], acc.at[idx_d.at[j]], add=True)

        @pl.when(j + 2 < NCHUNK)
        def _():
            pltpu.async_copy(p_hbm.at[idx_s.at[j + 2]], rows.at[0], sem0)

        pltpu.make_async_copy(p_hbm.at[idx_s.at[j + 1]], rows.at[1], sem1).wait()
        pltpu.sync_copy(rows.at[1], acc.at[idx_d.at[j + 1]], add=True)
        return carry

    lax.fori_loop(0, NCHUNK // 2, _main, 0)
    plsc.subcore_barrier()

    def _wb(k, carry):
        r = (s * 5 + k) * 128
        pltpu.sync_copy(acc.at[pl.ds(r, 128)], out_hbm.at[c, pl.ds(r, 128)])
        return carry

    lax.fori_loop(0, 5, _wb, 0)


@functools.lru_cache(maxsize=None)
def _scatter_kernel():
    return pl.kernel(
        _scatter_body,
        out_type=jax.ShapeDtypeStruct((2, NP, D), jnp.float32),
        mesh=_sc_mesh(),
        scratch_types=[
            pltpu.VMEM((NCHUNK, 128), jnp.int32),
            pltpu.VMEM((NCHUNK, 128), jnp.int32),
            pltpu.VMEM((2, 128, D), jnp.float32),
            pltpu.VMEM_SHARED((NP, D), jnp.float32),
            pltpu.SemaphoreType.DMA,
            pltpu.SemaphoreType.DMA,
        ],
    )


def _first_body(d0_ref, d1_ref, x_ref, w_ref, p_ref, dinv_ref):
    i = pl.program_id(0)
    deg = d0_ref[...] + d1_ref[...] + 1.0
    row = lax.broadcasted_iota(jnp.int32, (128, 1), 0) + i * 128
    dinv = jnp.where(row < N, lax.rsqrt(deg), 0.0)
    hw = jnp.dot(x_ref[...], w_ref[...],
                 preferred_element_type=jnp.float32, precision=_HIGHEST)
    p_ref[...] = dinv * hw
    dinv_ref[...] = dinv


def _first_call(d0, d1, x, w):
    return pl.pallas_call(
        _first_body,
        grid=(NP // 128,),
        in_specs=[
            pl.BlockSpec((128, 1), lambda i: (i, 0)),
            pl.BlockSpec((128, 1), lambda i: (i, 0)),
            pl.BlockSpec((128, D), lambda i: (i, 0)),
            pl.BlockSpec((D, D), lambda i: (0, 0)),
        ],
        out_specs=[
            pl.BlockSpec((128, D), lambda i: (i, 0)),
            pl.BlockSpec((128, 1), lambda i: (i, 0)),
        ],
        out_shape=[
            jax.ShapeDtypeStruct((NP, D), jnp.float32),
            jax.ShapeDtypeStruct((NP, 1), jnp.float32),
        ],
    )(d0, d1, x, w)


def _mid_body(s_ref, p_ref, dinv_ref, b_ref, w_ref, o_ref):
    dinv = dinv_ref[...]
    h = jnp.maximum(dinv * (s_ref[0] + s_ref[1] + p_ref[...]) + b_ref[...], 0.0)
    o_ref[...] = dinv * jnp.dot(h, w_ref[...],
                                preferred_element_type=jnp.float32,
                                precision=_HIGHEST)


def _mid_call(s, p, dinv, b, w):
    return pl.pallas_call(
        _mid_body,
        grid=(NP // 128,),
        in_specs=[
            pl.BlockSpec((2, 128, D), lambda i: (0, i, 0)),
            pl.BlockSpec((128, D), lambda i: (i, 0)),
            pl.BlockSpec((128, 1), lambda i: (i, 0)),
            pl.BlockSpec((1, D), lambda i: (0, 0)),
            pl.BlockSpec((D, D), lambda i: (0, 0)),
        ],
        out_specs=pl.BlockSpec((128, D), lambda i: (i, 0)),
        out_shape=jax.ShapeDtypeStruct((NP, D), jnp.float32),
    )(s, p, dinv, b, w)


def _final_body(s_ref, p_ref, dinv_ref, b_ref, batch_ref, o_ref, acc_ref, cnt_ref):
    i = pl.program_id(0)

    @pl.when(i == 0)
    def _():
        acc_ref[...] = jnp.zeros_like(acc_ref)
        cnt_ref[...] = jnp.zeros_like(cnt_ref)

    h = dinv_ref[...] * (s_ref[0] + s_ref[1] + p_ref[...]) + b_ref[...]
    gids = lax.broadcasted_iota(jnp.int32, (G, 1), 0)
    oh_t = (gids == batch_ref[0]).astype(jnp.float32)
    acc_ref[...] += jnp.dot(oh_t, h, preferred_element_type=jnp.float32,
                            precision=_HIGHEST)
    cnt_ref[...] += jnp.sum(oh_t, axis=1, keepdims=True)

    @pl.when(i == NP // 128 - 1)
    def _():
        o_ref[...] = acc_ref[...] / jnp.maximum(cnt_ref[...], 1.0)


def _final_call(s, p, dinv, b, batch3):
    return pl.pallas_call(
        _final_body,
        grid=(NP // 128,),
        in_specs=[
            pl.BlockSpec((2, 128, D), lambda i: (0, i, 0)),
            pl.BlockSpec((128, D), lambda i: (i, 0)),
            pl.BlockSpec((128, 1), lambda i: (i, 0)),
            pl.BlockSpec((1, D), lambda i: (0, 0)),
            pl.BlockSpec((1, 1, 128), lambda i: (i, 0, 0)),
        ],
        out_specs=pl.BlockSpec((G, D), lambda i: (0, 0)),
        out_shape=jax.ShapeDtypeStruct((G, D), jnp.float32),
        scratch_shapes=[
            pltpu.VMEM((G, D), jnp.float32),
            pltpu.VMEM((G, 1), jnp.float32),
        ],
    )(s, p, dinv, b, batch3)


def kernel(x, edge_index, batch, W1, b1, W2, b2, W3, b3):
    pad = jnp.full((EP - E,), N, dtype=jnp.int32)
    src3 = jnp.concatenate([edge_index[0], pad]).reshape(NW, NCHUNK, 128)
    dst3 = jnp.concatenate([edge_index[1], pad]).reshape(NW, NCHUNK, 128)
    xp = jnp.zeros((NP, D), jnp.float32).at[:N].set(x)
    batch3 = jnp.concatenate(
        [batch, jnp.full((NP - N,), G, dtype=jnp.int32)]).reshape(NP // 128, 1, 128)
    w3p = jnp.zeros((D, D), jnp.float32).at[:, :C].set(W3)
    b1r = b1.reshape(1, D)
    b2r = b2.reshape(1, D)
    b3r = jnp.zeros((1, D), jnp.float32).at[0, :C].set(b3)

    degs = _deg_kernel()(dst3)
    d0 = degs[0].reshape(NP, 1)
    d1 = degs[1].reshape(NP, 1)

    p1, dinv = _first_call(d0, d1, xp, W1)
    s1 = _scatter_kernel()(p1, src3, dst3)
    p2 = _mid_call(s1, p1, dinv, b1r, W2)
    s2 = _scatter_kernel()(p2, src3, dst3)
    p3 = _mid_call(s2, p2, dinv, b2r, w3p)
    s3 = _scatter_kernel()(p3, src3, dst3)
    out = _final_call(s3, p3, dinv, b3r, batch3)
    return out[:, :C]


# speedup vs baseline: 6.9786x; 6.9786x over previous
import functools

import jax
import jax.numpy as jnp
from jax import lax
from jax.experimental import pallas as pl
from jax.experimental.pallas import tpu as pltpu
from jax.experimental.pallas import tpu_sc as plsc

N = 10000
E = 320000
D = 128
C = 40
G = 64

NP = 10240
EP = 327680
NW = 32
NCHUNK = 80
ROWS_PER_TILE = NP // 16

_HIGHEST = jax.lax.Precision.HIGHEST


@functools.lru_cache(maxsize=None)
def _sc_mesh():
    return plsc.VectorSubcoreMesh(core_axis_name="c", subcore_axis_name="s")


def _deg_body(dst_hbm, out_hbm, idx_d, ones_v, zeros_v, acc):
    c = lax.axis_index("c")
    s = lax.axis_index("s")
    wid = s * 2 + c
    pltpu.sync_copy(dst_hbm.at[wid], idx_d)
    for k in range(8):
        ones_v[pl.ds(k * 16, 16)] = jnp.ones((16,), jnp.float32)

    def _zfill(t, carry):
        zeros_v[pl.ds(t * 16, 16)] = jnp.zeros((16,), jnp.float32)
        return carry

    lax.fori_loop(0, ROWS_PER_TILE // 16, _zfill, 0)
    pltpu.sync_copy(zeros_v, acc.at[pl.ds(s * ROWS_PER_TILE, ROWS_PER_TILE)])
    plsc.subcore_barrier()

    def _scat(j, carry):
        pltpu.sync_copy(ones_v, acc.at[idx_d.at[j]], add=True)
        return carry

    lax.fori_loop(0, NCHUNK, _scat, 0)
    plsc.subcore_barrier()
    pltpu.sync_copy(acc.at[pl.ds(s * ROWS_PER_TILE, ROWS_PER_TILE)],
                    out_hbm.at[c, pl.ds(s * ROWS_PER_TILE, ROWS_PER_TILE)])


@functools.lru_cache(maxsize=None)
def _deg_kernel():
    return pl.kernel(
        _deg_body,
        out_type=jax.ShapeDtypeStruct((2, NP), jnp.float32),
        mesh=_sc_mesh(),
        scratch_types=[
            pltpu.VMEM((NCHUNK, 128), jnp.int32),
            pltpu.VMEM((128,), jnp.float32),
            pltpu.VMEM((ROWS_PER_TILE,), jnp.float32),
            pltpu.VMEM_SHARED((NP,), jnp.float32),
        ],
    )


PH = 16


def _scatter_body(p_hbm, src_hbm, dst_hbm, out_hbm,
                  idx_s, idx_d, rows, acc, sem0, sem1):
    c = lax.axis_index("c")
    s = lax.axis_index("s")
    wid = s * 2 + c

    def _zrow(r, carry):
        for k in range(8):
            rows[0, r, pl.ds(k * 16, 16)] = jnp.zeros((16,), jnp.float32)
        return carry

    lax.fori_loop(0, 128, _zrow, 0)

    def _zcopy(k, carry):
        pltpu.sync_copy(rows.at[0],
                        acc.at[pl.ds((s * 5 + k) * 128, 128)])
        return carry

    lax.fori_loop(0, 5, _zcopy, 0)
    plsc.subcore_barrier()

    def _phase(ph, carry):
        pltpu.sync_copy(src_hbm.at[wid, pl.ds(ph * PH, PH)], idx_s)
        pltpu.sync_copy(dst_hbm.at[wid, pl.ds(ph * PH, PH)], idx_d)
        pltpu.async_copy(p_hbm.at[idx_s.at[0]], rows.at[0], sem0)

        def _main(t, carry2):
            j = t * 2
            pltpu.async_copy(p_hbm.at[idx_s.at[j + 1]], rows.at[1], sem1)
            pltpu.make_async_copy(
                p_hbm.at[idx_s.at[j]], rows.at[0], sem0).wait()
            pltpu.sync_copy(rows.at[0], acc.at[idx_d.at[j]], add=True)

            @pl.when(j + 2 < PH)
            def _():
                pltpu.async_copy(p_hbm.at[idx_s.at[j + 2]], rows.at[0], sem0)

            pltpu.make_async_copy(
                p_hbm.at[idx_s.at[j + 1]], rows.at[1], sem1).wait()
            pltpu.sync_copy(rows.at[1], acc.at[idx_d.at[j + 1]], add=True)
            return carry2

        lax.fori_loop(0, PH // 2, _main, 0)
        return carry

    lax.fori_loop(0, NCHUNK // PH, _phase, 0)
    plsc.subcore_barrier()

    def _wb(k, carry):
        r = (s * 5 + k) * 128
        pltpu.sync_copy(acc.at[pl.ds(r, 128)], out_hbm.at[c, pl.ds(r, 128)])
        return carry

    lax.fori_loop(0, 5, _wb, 0)


@functools.lru_cache(maxsize=None)
def _scatter_kernel():
    return pl.kernel(
        _scatter_body,
        out_type=jax.ShapeDtypeStruct((2, NP, D), jnp.float32),
        mesh=_sc_mesh(),
        scratch_types=[
            pltpu.VMEM((PH, 128), jnp.int32),
            pltpu.VMEM((PH, 128), jnp.int32),
            pltpu.VMEM((2, 128, D), jnp.float32),
            pltpu.VMEM_SHARED((NP, D), jnp.float32),
            pltpu.SemaphoreType.DMA,
            pltpu.SemaphoreType.DMA,
        ],
    )


def _first_body(d0_ref, d1_ref, x_ref, w_ref, p_ref, dinv_ref):
    i = pl.program_id(0)
    deg = d0_ref[...] + d1_ref[...] + 1.0
    row = lax.broadcasted_iota(jnp.int32, (128, 1), 0) + i * 128
    dinv = jnp.where(row < N, lax.rsqrt(deg), 0.0)
    hw = jnp.dot(x_ref[...], w_ref[...],
                 preferred_element_type=jnp.float32, precision=_HIGHEST)
    p_ref[...] = dinv * hw
    dinv_ref[...] = dinv


def _first_call(d0, d1, x, w):
    return pl.pallas_call(
        _first_body,
        grid=(NP // 128,),
        in_specs=[
            pl.BlockSpec((128, 1), lambda i: (i, 0)),
            pl.BlockSpec((128, 1), lambda i: (i, 0)),
            pl.BlockSpec((128, D), lambda i: (i, 0)),
            pl.BlockSpec((D, D), lambda i: (0, 0)),
        ],
        out_specs=[
            pl.BlockSpec((128, D), lambda i: (i, 0)),
            pl.BlockSpec((128, 1), lambda i: (i, 0)),
        ],
        out_shape=[
            jax.ShapeDtypeStruct((NP, D), jnp.float32),
            jax.ShapeDtypeStruct((NP, 1), jnp.float32),
        ],
    )(d0, d1, x, w)


def _mid_body(s_ref, p_ref, dinv_ref, b_ref, w_ref, o_ref):
    dinv = dinv_ref[...]
    h = jnp.maximum(dinv * (s_ref[0] + s_ref[1] + p_ref[...]) + b_ref[...], 0.0)
    o_ref[...] = dinv * jnp.dot(h, w_ref[...],
                                preferred_element_type=jnp.float32,
                                precision=_HIGHEST)


def _mid_call(s, p, dinv, b, w):
    return pl.pallas_call(
        _mid_body,
        grid=(NP // 128,),
        in_specs=[
            pl.BlockSpec((2, 128, D), lambda i: (0, i, 0)),
            pl.BlockSpec((128, D), lambda i: (i, 0)),
            pl.BlockSpec((128, 1), lambda i: (i, 0)),
            pl.BlockSpec((1, D), lambda i: (0, 0)),
            pl.BlockSpec((D, D), lambda i: (0, 0)),
        ],
        out_specs=pl.BlockSpec((128, D), lambda i: (i, 0)),
        out_shape=jax.ShapeDtypeStruct((NP, D), jnp.float32),
    )(s, p, dinv, b, w)


def _final_body(s_ref, p_ref, dinv_ref, b_ref, batch_ref, o_ref, acc_ref, cnt_ref):
    i = pl.program_id(0)

    @pl.when(i == 0)
    def _():
        acc_ref[...] = jnp.zeros_like(acc_ref)
        cnt_ref[...] = jnp.zeros_like(cnt_ref)

    h = dinv_ref[...] * (s_ref[0] + s_ref[1] + p_ref[...]) + b_ref[...]
    gids = lax.broadcasted_iota(jnp.int32, (G, 1), 0)
    oh_t = (gids == batch_ref[0]).astype(jnp.float32)
    acc_ref[...] += jnp.dot(oh_t, h, preferred_element_type=jnp.float32,
                            precision=_HIGHEST)
    cnt_ref[...] += jnp.sum(oh_t, axis=1, keepdims=True)

    @pl.when(i == NP // 128 - 1)
    def _():
        o_ref[...] = acc_ref[...] / jnp.maximum(cnt_ref[...], 1.0)


def _final_call(s, p, dinv, b, batch3):
    return pl.pallas_call(
        _final_body,
        grid=(NP // 128,),
        in_specs=[
            pl.BlockSpec((2, 128, D), lambda i: (0, i, 0)),
            pl.BlockSpec((128, D), lambda i: (i, 0)),
            pl.BlockSpec((128, 1), lambda i: (i, 0)),
            pl.BlockSpec((1, D), lambda i: (0, 0)),
            pl.BlockSpec((1, 1, 128), lambda i: (i, 0, 0)),
        ],
        out_specs=pl.BlockSpec((G, D), lambda i: (0, 0)),
        out_shape=jax.ShapeDtypeStruct((G, D), jnp.float32),
        scratch_shapes=[
            pltpu.VMEM((G, D), jnp.float32),
            pltpu.VMEM((G, 1), jnp.float32),
        ],
    )(s, p, dinv, b, batch3)


def kernel(x, edge_index, batch, W1, b1, W2, b2, W3, b3):
    pad = jnp.full((EP - E,), N, dtype=jnp.int32)
    src3 = jnp.concatenate([edge_index[0], pad]).reshape(NW, NCHUNK, 128)
    dst3 = jnp.concatenate([edge_index[1], pad]).reshape(NW, NCHUNK, 128)
    xp = jnp.zeros((NP, D), jnp.float32).at[:N].set(x)
    batch3 = jnp.concatenate(
        [batch, jnp.full((NP - N,), G, dtype=jnp.int32)]).reshape(NP // 128, 1, 128)
    w3p = jnp.zeros((D, D), jnp.float32).at[:, :C].set(W3)
    b1r = b1.reshape(1, D)
    b2r = b2.reshape(1, D)
    b3r = jnp.zeros((1, D), jnp.float32).at[0, :C].set(b3)

    degs = _deg_kernel()(dst3)
    d0 = degs[0].reshape(NP, 1)
    d1 = degs[1].reshape(NP, 1)

    p1, dinv = _first_call(d0, d1, xp, W1)
    s1 = _scatter_kernel()(p1, src3, dst3)
    p2 = _mid_call(s1, p1, dinv, b1r, W2)
    s2 = _scatter_kernel()(p2, src3, dst3)
    p3 = _mid_call(s2, p2, dinv, b2r, w3p)
    s3 = _scatter_kernel()(p3, src3, dst3)
    out = _final_call(s3, p3, dinv, b3r, batch3)
    return out[:, :C]
